# baseline (speedup 1.0000x reference)
import functools

import jax
import jax.numpy as jnp
from jax import lax
from jax.experimental import pallas as pl
from jax.experimental.pallas import tpu as pltpu
from jax.experimental.pallas import tpu_sc as plsc

N = 10000
E = 320000
H = 128
G = 128

NC = 2
NS = 16
NW = NC * NS
CHUNK = 128
EPW = E // NW
NCHUNK = EPW // CHUNK
TAIL = EPW - NCHUNK * CHUNK
SPR = 624
_STRIPE = (128, 128, 128, 128, 112)
TAILR = N - NS * SPR

BN = 1000

_mesh = plsc.VectorSubcoreMesh(core_axis_name="c", subcore_axis_name="s")


def _zero_acc(sid, buf, acc):
    r0 = sid * SPR
    off = 0
    for nr in _STRIPE:
        pltpu.sync_copy(buf.at[pl.ds(0, nr)], acc.at[pl.ds(r0 + off, nr)])
        off += nr

    @pl.when(sid == 0)
    def _():
        pltpu.sync_copy(buf.at[pl.ds(0, TAILR)], acc.at[pl.ds(NS * SPR, TAILR)])


def _drain_acc(sid, cid, acc, buf, out_hbm):
    r0 = sid * SPR
    off = 0
    for nr in _STRIPE:
        pltpu.sync_copy(acc.at[pl.ds(r0 + off, nr)], buf.at[pl.ds(0, nr)])
        pltpu.sync_copy(buf.at[pl.ds(0, nr)],
                        out_hbm.at[cid, pl.ds(r0 + off, nr)])
        off += nr

    @pl.when(sid == 0)
    def _():
        pltpu.sync_copy(acc.at[pl.ds(NS * SPR, TAILR)], buf.at[pl.ds(0, TAILR)])
        pltpu.sync_copy(buf.at[pl.ds(0, TAILR)],
                        out_hbm.at[cid, pl.ds(NS * SPR, TAILR)])


NCH2 = 80
CPW = NCH2 + 2
NACC = N + 8


@functools.partial(
    pl.kernel,
    mesh=_mesh,
    out_type=jax.ShapeDtypeStruct((NC, N, H), jnp.float32),
    scratch_types=[
        pltpu.VMEM((CHUNK,), jnp.int32),
        pltpu.VMEM((CHUNK,), jnp.int32),
        pltpu.VMEM((CHUNK,), jnp.int32),
        pltpu.VMEM((CHUNK,), jnp.int32),
        pltpu.VMEM((CHUNK, H), jnp.float32),
        pltpu.VMEM((CHUNK, H), jnp.float32),
        pltpu.VMEM_SHARED((NACC, H), jnp.float32),
        pltpu.SemaphoreType.DMA,
        pltpu.SemaphoreType.DMA,
    ],
)
def _msgpass(h_hbm, src_hbm, dst_hbm, out_hbm, src_a, dst_a, src_b, dst_b,
             rows_a, rows_b, acc, sem_a, sem_b):
    cid = lax.axis_index("c")
    sid = lax.axis_index("s")
    wid = sid * NC + cid
    base0 = wid * (CPW * CHUNK)

    def stage(c, src_buf, dst_buf):
        off = pl.multiple_of(base0 + c * CHUNK, 8)
        pltpu.sync_copy(src_hbm.at[pl.ds(off, CHUNK)], src_buf)
        pltpu.sync_copy(dst_hbm.at[pl.ds(off, CHUNK)], dst_buf)

    z16 = jnp.zeros((16,), jnp.float32)
    for r in range(CHUNK):
        for c in range(H // 16):
            rows_a[r, pl.ds(c * 16, 16)] = z16
    _zero_acc(sid, rows_a, acc)

    @pl.when(sid == 0)
    def _():
        pltpu.sync_copy(rows_a.at[pl.ds(0, NACC - N)], acc.at[pl.ds(N, NACC - N)])
    plsc.subcore_barrier()

    stage(0, src_a, dst_a)
    pltpu.async_copy(h_hbm.at[src_a], rows_a, sem_a)
    stage(1, src_b, dst_b)

    def body(k, carry):
        c0 = 2 * k
        pltpu.make_async_copy(h_hbm.at[src_a], rows_a, sem_a).wait()
        pltpu.async_copy(h_hbm.at[src_b], rows_b, sem_b)
        pltpu.sync_copy(rows_a, acc.at[dst_a], add=True)
        stage(c0 + 2, src_a, dst_a)
        pltpu.make_async_copy(h_hbm.at[src_b], rows_b, sem_b).wait()
        pltpu.async_copy(h_hbm.at[src_a], rows_a, sem_a)
        pltpu.sync_copy(rows_b, acc.at[dst_b], add=True)
        stage(c0 + 3, src_b, dst_b)
        return carry

    lax.fori_loop(0, NCH2 // 2, body, 0)
    pltpu.make_async_copy(h_hbm.at[src_a], rows_a, sem_a).wait()
    plsc.subcore_barrier()

    _drain_acc(sid, cid, acc, rows_a, out_hbm)


DROWS = 80
BE = 4000
DSTEPS = E // BE


def _deg_body(dst_ref, o_ref):
    i = pl.program_id(0)
    d = dst_ref[...]
    row = lax.shift_right_logical(d, 7)
    col = jnp.bitwise_and(d, 127)
    r_oh = (row == lax.broadcasted_iota(jnp.int32, (BE, DROWS), 1)
            ).astype(jnp.float32)
    c_oh = (col == lax.broadcasted_iota(jnp.int32, (BE, 128), 1)
            ).astype(jnp.float32)
    p = lax.dot_general(r_oh, c_oh, (((0,), (0,)), ((), ())),
                        preferred_element_type=jnp.float32,
                precision=lax.Precision.HIGHEST)

    @pl.when(i == 0)
    def _init():
        o_ref[...] = p

    @pl.when(i != 0)
    def _accum():
        o_ref[...] = o_ref[...] + p


_degrees = pl.pallas_call(
    _deg_body,
    grid=(DSTEPS,),
    in_specs=[pl.BlockSpec((BE, 1), lambda i: (i, 0))],
    out_specs=pl.BlockSpec((DROWS, 128), lambda i: (0, 0)),
    out_shape=jax.ShapeDtypeStruct((DROWS, 128), jnp.float32),
)


def _enc_body(x_ref, w_ref, b_ref, o_ref):
    o_ref[...] = (
        jnp.dot(x_ref[...], w_ref[...], preferred_element_type=jnp.float32)
        + b_ref[...])


_enc = pl.pallas_call(
    _enc_body,
    grid=(N // BN,),
    in_specs=[
        pl.BlockSpec((BN, H), lambda i: (i, 0)),
        pl.BlockSpec((H, H), lambda i: (0, 0)),
        pl.BlockSpec((1, H), lambda i: (0, 0)),
    ],
    out_specs=pl.BlockSpec((BN, H), lambda i: (i, 0)),
    out_shape=jax.ShapeDtypeStruct((N, H), jnp.float32),
)


def _combine_body(aggp_ref, deg_ref, h_ref, wl_ref, bl_ref, wr_ref, o_ref):
    inv = 1.0 / jnp.maximum(deg_ref[...], 1.0)
    agg = (aggp_ref[0] + aggp_ref[1]) * inv
    o_ref[...] = jnp.maximum(
        jnp.dot(agg, wl_ref[...], preferred_element_type=jnp.float32)
        + bl_ref[...]
        + jnp.dot(h_ref[...], wr_ref[...], preferred_element_type=jnp.float32),
        0.0)


_combine = pl.pallas_call(
    _combine_body,
    grid=(N // BN,),
    in_specs=[
        pl.BlockSpec((NC, BN, H), lambda i: (0, i, 0)),
        pl.BlockSpec((BN, 1), lambda i: (i, 0)),
        pl.BlockSpec((BN, H), lambda i: (i, 0)),
        pl.BlockSpec((H, H), lambda i: (0, 0)),
        pl.BlockSpec((1, H), lambda i: (0, 0)),
        pl.BlockSpec((H, H), lambda i: (0, 0)),
    ],
    out_specs=pl.BlockSpec((BN, H), lambda i: (i, 0)),
    out_shape=jax.ShapeDtypeStruct((N, H), jnp.float32),
)


PCH = N // CHUNK
PTAIL = N - PCH * CHUNK


@functools.partial(
    pl.kernel,
    mesh=_mesh,
    out_type=jax.ShapeDtypeStruct((NC, G, H), jnp.float32),
    scratch_types=[
        pltpu.VMEM((CHUNK,), jnp.int32),
        pltpu.VMEM((CHUNK, H), jnp.float32),
        pltpu.VMEM((PTAIL,), jnp.int32),
        pltpu.VMEM((PTAIL, H), jnp.float32),
        pltpu.VMEM_SHARED((G, H), jnp.float32),
        pltpu.SemaphoreType.DMA,
    ],
)
def _pool(h_hbm, batch_hbm, out_hbm, bidx_v, rows_v, bidx_t, rows_t, acc, sem):
    cid = lax.axis_index("c")
    sid = lax.axis_index("s")
    wid = sid * NC + cid

    z16 = jnp.zeros((16,), jnp.float32)
    for r in range(CHUNK):
        for c in range(H // 16):
            rows_v[r, pl.ds(c * 16, 16)] = z16

    @pl.when(sid == 0)
    def _():
        pltpu.sync_copy(rows_v, acc)
    plsc.subcore_barrier()

    for rep in range(3):
        ch = wid + NW * rep

        @pl.when(ch < PCH)
        def _():
            off = pl.multiple_of(ch * CHUNK, 8)
            pltpu.sync_copy(batch_hbm.at[pl.ds(off, CHUNK)], bidx_v)
            pltpu.sync_copy(h_hbm.at[pl.ds(off, CHUNK)], rows_v)
            pltpu.sync_copy(rows_v, acc.at[bidx_v], add=True)

    @pl.when(wid == NW - 1)
    def _():
        pltpu.sync_copy(batch_hbm.at[pl.ds(PCH * CHUNK, PTAIL)], bidx_t)
        pltpu.sync_copy(h_hbm.at[pl.ds(PCH * CHUNK, PTAIL)], rows_t)
        pltpu.sync_copy(rows_t, acc.at[bidx_t], add=True)
    plsc.subcore_barrier()

    @pl.when(sid == 0)
    def _():
        pltpu.sync_copy(acc, rows_v)
        pltpu.sync_copy(rows_v, out_hbm.at[cid])


def _readout_body(p_ref, wo_ref, bo_ref, o_ref):
    pooled = p_ref[0] + p_ref[1]
    o_ref[...] = jnp.dot(pooled, wo_ref[...],
                         preferred_element_type=jnp.float32) + bo_ref[...]


_readout = pl.pallas_call(
    _readout_body,
    grid=(1,),
    in_specs=[
        pl.BlockSpec((NC, G, H), lambda i: (0, 0, 0)),
        pl.BlockSpec((H, 1), lambda i: (0, 0)),
        pl.BlockSpec((1, 1), lambda i: (0, 0)),
    ],
    out_specs=pl.BlockSpec((G, 1), lambda i: (0, 0)),
    out_shape=jax.ShapeDtypeStruct((G, 1), jnp.float32),
)


def kernel(x, edge_index, batch, W_enc, b_enc, W_out, b_out,
           W_l0, b_l0, W_r0, W_l1, b_l1, W_r1, W_l2, b_l2, W_r2):
    src, dst = edge_index[0], edge_index[1]
    epad = NW * NCH2 * CHUNK - E
    srcp = jnp.concatenate([src, jnp.zeros((epad,), jnp.int32)])
    dstp = jnp.concatenate([dst, jnp.full((epad,), N, jnp.int32)])
    srcp = jnp.pad(srcp.reshape(NW, NCH2, CHUNK),
                   ((0, 0), (0, CPW - NCH2), (0, 0))).reshape(-1)
    dstp = jnp.pad(dstp.reshape(NW, NCH2, CHUNK),
                   ((0, 0), (0, CPW - NCH2), (0, 0)),
                   constant_values=N).reshape(-1)
    deg = _degrees(dst.reshape(E, 1))
    deg = deg.reshape(DROWS * 128)[:N].reshape(N, 1)
    h = _enc(x, W_enc, b_enc.reshape(1, H))
    for (W_l, b_l, W_r) in ((W_l0, b_l0, W_r0), (W_l1, b_l1, W_r1)):
        aggp = _msgpass(h, srcp, dstp)
        h = _combine(aggp, deg, h, W_l, b_l.reshape(1, H), W_r)
    aggp = _msgpass(h, srcp, dstp)
    h3 = _combine(aggp, deg, h, W_l2, b_l2.reshape(1, H), W_r2)
    pooled = _pool(h3, batch)
    return _readout(pooled, W_out, b_out.reshape(1, 1))


# speedup vs baseline: 1.1071x; 1.1071x over previous
import functools

import jax
import jax.numpy as jnp
from jax import lax
from jax.experimental import pallas as pl
from jax.experimental.pallas import tpu as pltpu
from jax.experimental.pallas import tpu_sc as plsc

N = 10000
E = 320000
H = 128
G = 128

NC = 2
NS = 16
NW = NC * NS
CHUNK = 128
EPW = E // NW
NCHUNK = EPW // CHUNK
TAIL = EPW - NCHUNK * CHUNK
SPR = 624
_STRIPE = (128, 128, 128, 128, 112)
TAILR = N - NS * SPR

BN = 1000

_mesh = plsc.VectorSubcoreMesh(core_axis_name="c", subcore_axis_name="s")


def _zero_acc(sid, buf, acc):
    r0 = sid * SPR
    off = 0
    for nr in _STRIPE:
        pltpu.sync_copy(buf.at[pl.ds(0, nr)], acc.at[pl.ds(r0 + off, nr)])
        off += nr

    @pl.when(sid == 0)
    def _():
        pltpu.sync_copy(buf.at[pl.ds(0, TAILR)], acc.at[pl.ds(NS * SPR, TAILR)])


def _drain_acc(sid, cid, acc, buf, out_hbm):
    r0 = sid * SPR
    off = 0
    for nr in _STRIPE:
        pltpu.sync_copy(acc.at[pl.ds(r0 + off, nr)], buf.at[pl.ds(0, nr)])
        pltpu.sync_copy(buf.at[pl.ds(0, nr)],
                        out_hbm.at[cid, pl.ds(r0 + off, nr)])
        off += nr

    @pl.when(sid == 0)
    def _():
        pltpu.sync_copy(acc.at[pl.ds(NS * SPR, TAILR)], buf.at[pl.ds(0, TAILR)])
        pltpu.sync_copy(buf.at[pl.ds(0, TAILR)],
                        out_hbm.at[cid, pl.ds(NS * SPR, TAILR)])


NCH2 = 80
CPW = NCH2 + 2
NACC = N + 8


@functools.partial(
    pl.kernel,
    mesh=_mesh,
    out_type=jax.ShapeDtypeStruct((NC, N, H), jnp.float32),
    scratch_types=[
        pltpu.VMEM((CHUNK,), jnp.int32),
        pltpu.VMEM((CHUNK,), jnp.int32),
        pltpu.VMEM((CHUNK,), jnp.int32),
        pltpu.VMEM((CHUNK,), jnp.int32),
        pltpu.VMEM((CHUNK, H), jnp.float32),
        pltpu.VMEM((CHUNK, H), jnp.float32),
        pltpu.VMEM_SHARED((NACC, H), jnp.float32),
        pltpu.SemaphoreType.DMA,
        pltpu.SemaphoreType.DMA,
    ],
)
def _msgpass(h_hbm, src_hbm, dst_hbm, out_hbm, src_a, dst_a, src_b, dst_b,
             rows_a, rows_b, acc, sem_a, sem_b):
    cid = lax.axis_index("c")
    sid = lax.axis_index("s")
    wid = sid * NC + cid
    base0 = wid * (CPW * CHUNK)

    def stage(c, src_buf, dst_buf):
        off = pl.multiple_of(base0 + c * CHUNK, 8)
        pltpu.sync_copy(src_hbm.at[pl.ds(off, CHUNK)], src_buf)
        pltpu.sync_copy(dst_hbm.at[pl.ds(off, CHUNK)], dst_buf)

    z16 = jnp.zeros((16,), jnp.float32)
    for r in range(CHUNK):
        for c in range(H // 16):
            rows_a[r, pl.ds(c * 16, 16)] = z16
    _zero_acc(sid, rows_a, acc)

    @pl.when(sid == 0)
    def _():
        pltpu.sync_copy(rows_a.at[pl.ds(0, NACC - N)], acc.at[pl.ds(N, NACC - N)])
    plsc.subcore_barrier()

    def body(k, carry):
        stage(k, src_a, dst_a)
        pltpu.async_copy(h_hbm.at[src_a], rows_a, sem_a).wait()
        pltpu.sync_copy(rows_a, acc.at[dst_a], add=True)
        return carry

    lax.fori_loop(0, NCH2, body, 0)
    plsc.subcore_barrier()

    _drain_acc(sid, cid, acc, rows_a, out_hbm)


DROWS = 80
BE = 4000
DSTEPS = E // BE


def _deg_body(dst_ref, o_ref):
    i = pl.program_id(0)
    d = dst_ref[...]
    row = lax.shift_right_logical(d, 7)
    col = jnp.bitwise_and(d, 127)
    r_oh = (row == lax.broadcasted_iota(jnp.int32, (BE, DROWS), 1)
            ).astype(jnp.float32)
    c_oh = (col == lax.broadcasted_iota(jnp.int32, (BE, 128), 1)
            ).astype(jnp.float32)
    p = lax.dot_general(r_oh, c_oh, (((0,), (0,)), ((), ())),
                        preferred_element_type=jnp.float32,
                precision=lax.Precision.HIGHEST)

    @pl.when(i == 0)
    def _init():
        o_ref[...] = p

    @pl.when(i != 0)
    def _accum():
        o_ref[...] = o_ref[...] + p


_degrees = pl.pallas_call(
    _deg_body,
    grid=(DSTEPS,),
    in_specs=[pl.BlockSpec((BE, 1), lambda i: (i, 0))],
    out_specs=pl.BlockSpec((DROWS, 128), lambda i: (0, 0)),
    out_shape=jax.ShapeDtypeStruct((DROWS, 128), jnp.float32),
)


def _enc_body(x_ref, w_ref, b_ref, o_ref):
    o_ref[...] = (
        jnp.dot(x_ref[...], w_ref[...], preferred_element_type=jnp.float32)
        + b_ref[...])


_enc = pl.pallas_call(
    _enc_body,
    grid=(N // BN,),
    in_specs=[
        pl.BlockSpec((BN, H), lambda i: (i, 0)),
        pl.BlockSpec((H, H), lambda i: (0, 0)),
        pl.BlockSpec((1, H), lambda i: (0, 0)),
    ],
    out_specs=pl.BlockSpec((BN, H), lambda i: (i, 0)),
    out_shape=jax.ShapeDtypeStruct((N, H), jnp.float32),
)


def _combine_body(aggp_ref, deg_ref, h_ref, wl_ref, bl_ref, wr_ref, o_ref):
    inv = 1.0 / jnp.maximum(deg_ref[...], 1.0)
    agg = (aggp_ref[0] + aggp_ref[1]) * inv
    o_ref[...] = jnp.maximum(
        jnp.dot(agg, wl_ref[...], preferred_element_type=jnp.float32)
        + bl_ref[...]
        + jnp.dot(h_ref[...], wr_ref[...], preferred_element_type=jnp.float32),
        0.0)


_combine = pl.pallas_call(
    _combine_body,
    grid=(N // BN,),
    in_specs=[
        pl.BlockSpec((NC, BN, H), lambda i: (0, i, 0)),
        pl.BlockSpec((BN, 1), lambda i: (i, 0)),
        pl.BlockSpec((BN, H), lambda i: (i, 0)),
        pl.BlockSpec((H, H), lambda i: (0, 0)),
        pl.BlockSpec((1, H), lambda i: (0, 0)),
        pl.BlockSpec((H, H), lambda i: (0, 0)),
    ],
    out_specs=pl.BlockSpec((BN, H), lambda i: (i, 0)),
    out_shape=jax.ShapeDtypeStruct((N, H), jnp.float32),
)


PCH = N // CHUNK
PTAIL = N - PCH * CHUNK


@functools.partial(
    pl.kernel,
    mesh=_mesh,
    out_type=jax.ShapeDtypeStruct((NC, G, H), jnp.float32),
    scratch_types=[
        pltpu.VMEM((CHUNK,), jnp.int32),
        pltpu.VMEM((CHUNK, H), jnp.float32),
        pltpu.VMEM((PTAIL,), jnp.int32),
        pltpu.VMEM((PTAIL, H), jnp.float32),
        pltpu.VMEM_SHARED((G, H), jnp.float32),
        pltpu.SemaphoreType.DMA,
    ],
)
def _pool(h_hbm, batch_hbm, out_hbm, bidx_v, rows_v, bidx_t, rows_t, acc, sem):
    cid = lax.axis_index("c")
    sid = lax.axis_index("s")
    wid = sid * NC + cid

    z16 = jnp.zeros((16,), jnp.float32)
    for r in range(CHUNK):
        for c in range(H // 16):
            rows_v[r, pl.ds(c * 16, 16)] = z16

    @pl.when(sid == 0)
    def _():
        pltpu.sync_copy(rows_v, acc)
    plsc.subcore_barrier()

    for rep in range(3):
        ch = wid + NW * rep

        @pl.when(ch < PCH)
        def _():
            off = pl.multiple_of(ch * CHUNK, 8)
            pltpu.sync_copy(batch_hbm.at[pl.ds(off, CHUNK)], bidx_v)
            pltpu.sync_copy(h_hbm.at[pl.ds(off, CHUNK)], rows_v)
            pltpu.sync_copy(rows_v, acc.at[bidx_v], add=True)

    @pl.when(wid == NW - 1)
    def _():
        pltpu.sync_copy(batch_hbm.at[pl.ds(PCH * CHUNK, PTAIL)], bidx_t)
        pltpu.sync_copy(h_hbm.at[pl.ds(PCH * CHUNK, PTAIL)], rows_t)
        pltpu.sync_copy(rows_t, acc.at[bidx_t], add=True)
    plsc.subcore_barrier()

    @pl.when(sid == 0)
    def _():
        pltpu.sync_copy(acc, rows_v)
        pltpu.sync_copy(rows_v, out_hbm.at[cid])


def _readout_body(p_ref, wo_ref, bo_ref, o_ref):
    pooled = p_ref[0] + p_ref[1]
    o_ref[...] = jnp.dot(pooled, wo_ref[...],
                         preferred_element_type=jnp.float32) + bo_ref[...]


_readout = pl.pallas_call(
    _readout_body,
    grid=(1,),
    in_specs=[
        pl.BlockSpec((NC, G, H), lambda i: (0, 0, 0)),
        pl.BlockSpec((H, 1), lambda i: (0, 0)),
        pl.BlockSpec((1, 1), lambda i: (0, 0)),
    ],
    out_specs=pl.BlockSpec((G, 1), lambda i: (0, 0)),
    out_shape=jax.ShapeDtypeStruct((G, 1), jnp.float32),
)


def kernel(x, edge_index, batch, W_enc, b_enc, W_out, b_out,
           W_l0, b_l0, W_r0, W_l1, b_l1, W_r1, W_l2, b_l2, W_r2):
    src, dst = edge_index[0], edge_index[1]
    epad = NW * NCH2 * CHUNK - E
    srcp = jnp.concatenate([src, jnp.zeros((epad,), jnp.int32)])
    dstp = jnp.concatenate([dst, jnp.full((epad,), N, jnp.int32)])
    srcp = jnp.pad(srcp.reshape(NW, NCH2, CHUNK),
                   ((0, 0), (0, CPW - NCH2), (0, 0))).reshape(-1)
    dstp = jnp.pad(dstp.reshape(NW, NCH2, CHUNK),
                   ((0, 0), (0, CPW - NCH2), (0, 0)),
                   constant_values=N).reshape(-1)
    deg = _degrees(dst.reshape(E, 1))
    deg = deg.reshape(DROWS * 128)[:N].reshape(N, 1)
    h = _enc(x, W_enc, b_enc.reshape(1, H))
    for (W_l, b_l, W_r) in ((W_l0, b_l0, W_r0), (W_l1, b_l1, W_r1)):
        aggp = _msgpass(h, srcp, dstp)
        h = _combine(aggp, deg, h, W_l, b_l.reshape(1, H), W_r)
    aggp = _msgpass(h, srcp, dstp)
    h3 = _combine(aggp, deg, h, W_l2, b_l2.reshape(1, H), W_r2)
    pooled = _pool(h3, batch)
    return _readout(pooled, W_out, b_out.reshape(1, 1))


# speedup vs baseline: 2.4024x; 2.1700x over previous
import functools

import jax
import jax.numpy as jnp
from jax import lax
from jax.experimental import pallas as pl
from jax.experimental.pallas import tpu as pltpu
from jax.experimental.pallas import tpu_sc as plsc

N = 10000
E = 320000
H = 128
G = 128

NC = 2
NS = 16
NW = NC * NS
CHUNK = 128
EPW = E // NW
NCHUNK = EPW // CHUNK
TAIL = EPW - NCHUNK * CHUNK
SPR = 624
_STRIPE = (128, 128, 128, 128, 112)
TAILR = N - NS * SPR

BN = 1000

_mesh = plsc.VectorSubcoreMesh(core_axis_name="c", subcore_axis_name="s")


def _zero_acc(sid, buf, acc):
    r0 = sid * SPR
    off = 0
    for nr in _STRIPE:
        pltpu.sync_copy(buf.at[pl.ds(0, nr)], acc.at[pl.ds(r0 + off, nr)])
        off += nr

    @pl.when(sid == 0)
    def _():
        pltpu.sync_copy(buf.at[pl.ds(0, TAILR)], acc.at[pl.ds(NS * SPR, TAILR)])


def _drain_acc(sid, cid, acc, buf, out_hbm):
    r0 = sid * SPR
    off = 0
    for nr in _STRIPE:
        pltpu.sync_copy(acc.at[pl.ds(r0 + off, nr)], buf.at[pl.ds(0, nr)])
        pltpu.sync_copy(buf.at[pl.ds(0, nr)],
                        out_hbm.at[cid, pl.ds(r0 + off, nr)])
        off += nr

    @pl.when(sid == 0)
    def _():
        pltpu.sync_copy(acc.at[pl.ds(NS * SPR, TAILR)], buf.at[pl.ds(0, TAILR)])
        pltpu.sync_copy(buf.at[pl.ds(0, TAILR)],
                        out_hbm.at[cid, pl.ds(NS * SPR, TAILR)])


@functools.partial(
    pl.kernel,
    mesh=_mesh,
    out_type=jax.ShapeDtypeStruct((NC, N, H), jnp.float32),
    scratch_types=[
        pltpu.VMEM((CHUNK,), jnp.int32),
        pltpu.VMEM((CHUNK,), jnp.int32),
        pltpu.VMEM((CHUNK, H), jnp.float32),
        pltpu.VMEM((TAIL,), jnp.int32),
        pltpu.VMEM((TAIL,), jnp.int32),
        pltpu.VMEM((TAIL, H), jnp.float32),
        pltpu.VMEM_SHARED((N, H), jnp.float32),
        pltpu.SemaphoreType.DMA,
    ],
)
def _msgpass(h_hbm, ei_hbm, out_hbm, src_v, dst_v, rows_v, src_t, dst_t,
             rows_t, acc, sem):
    cid = lax.axis_index("c")
    sid = lax.axis_index("s")
    wid = sid * NC + cid

    z16 = jnp.zeros((16,), jnp.float32)
    for r in range(CHUNK):
        for c in range(H // 16):
            rows_v[r, pl.ds(c * 16, 16)] = z16
    _zero_acc(sid, rows_v, acc)
    plsc.subcore_barrier()

    base0 = wid * EPW

    def body(k, carry):
        base = pl.multiple_of(base0 + k * CHUNK, 8)
        pltpu.sync_copy(ei_hbm.at[pl.ds(base, CHUNK)], src_v)
        pltpu.sync_copy(ei_hbm.at[pl.ds(E + base, CHUNK)], dst_v)
        pltpu.async_copy(h_hbm.at[src_v], rows_v, sem).wait()
        pltpu.sync_copy(rows_v, acc.at[dst_v], add=True)
        return carry

    lax.fori_loop(0, NCHUNK, body, 0)

    baset = pl.multiple_of(base0 + NCHUNK * CHUNK, 8)
    pltpu.sync_copy(ei_hbm.at[pl.ds(baset, TAIL)], src_t)
    pltpu.sync_copy(ei_hbm.at[pl.ds(E + baset, TAIL)], dst_t)
    pltpu.async_copy(h_hbm.at[src_t], rows_t, sem).wait()
    pltpu.sync_copy(rows_t, acc.at[dst_t], add=True)
    plsc.subcore_barrier()

    _drain_acc(sid, cid, acc, rows_v, out_hbm)


DROWS = 80
BE = 4000
DSTEPS = E // BE


def _deg_body(dst_ref, o_ref):
    i = pl.program_id(0)
    d = dst_ref[...]
    row = lax.shift_right_logical(d, 7)
    col = jnp.bitwise_and(d, 127)
    r_oh = (row == lax.broadcasted_iota(jnp.int32, (BE, DROWS), 1)
            ).astype(jnp.float32)
    c_oh = (col == lax.broadcasted_iota(jnp.int32, (BE, 128), 1)
            ).astype(jnp.float32)
    p = lax.dot_general(r_oh, c_oh, (((0,), (0,)), ((), ())),
                        preferred_element_type=jnp.float32,
                precision=lax.Precision.HIGHEST)

    @pl.when(i == 0)
    def _init():
        o_ref[...] = p

    @pl.when(i != 0)
    def _accum():
        o_ref[...] = o_ref[...] + p


_degrees = pl.pallas_call(
    _deg_body,
    grid=(DSTEPS,),
    in_specs=[pl.BlockSpec((BE, 1), lambda i: (i, 0))],
    out_specs=pl.BlockSpec((DROWS, 128), lambda i: (0, 0)),
    out_shape=jax.ShapeDtypeStruct((DROWS, 128), jnp.float32),
)


def _enc_body(x_ref, w_ref, b_ref, o_ref):
    o_ref[...] = (
        jnp.dot(x_ref[...], w_ref[...], preferred_element_type=jnp.float32)
        + b_ref[...])


_enc = pl.pallas_call(
    _enc_body,
    grid=(N // BN,),
    in_specs=[
        pl.BlockSpec((BN, H), lambda i: (i, 0)),
        pl.BlockSpec((H, H), lambda i: (0, 0)),
        pl.BlockSpec((1, H), lambda i: (0, 0)),
    ],
    out_specs=pl.BlockSpec((BN, H), lambda i: (i, 0)),
    out_shape=jax.ShapeDtypeStruct((N, H), jnp.float32),
)


def _combine_body(aggp_ref, deg_ref, h_ref, wl_ref, bl_ref, wr_ref, o_ref):
    inv = 1.0 / jnp.maximum(deg_ref[...], 1.0)
    agg = (aggp_ref[0] + aggp_ref[1]) * inv
    o_ref[...] = jnp.maximum(
        jnp.dot(agg, wl_ref[...], preferred_element_type=jnp.float32)
        + bl_ref[...]
        + jnp.dot(h_ref[...], wr_ref[...], preferred_element_type=jnp.float32),
        0.0)


_combine = pl.pallas_call(
    _combine_body,
    grid=(N // BN,),
    in_specs=[
        pl.BlockSpec((NC, BN, H), lambda i: (0, i, 0)),
        pl.BlockSpec((BN, 1), lambda i: (i, 0)),
        pl.BlockSpec((BN, H), lambda i: (i, 0)),
        pl.BlockSpec((H, H), lambda i: (0, 0)),
        pl.BlockSpec((1, H), lambda i: (0, 0)),
        pl.BlockSpec((H, H), lambda i: (0, 0)),
    ],
    out_specs=pl.BlockSpec((BN, H), lambda i: (i, 0)),
    out_shape=jax.ShapeDtypeStruct((N, H), jnp.float32),
)


PCH = N // CHUNK
PTAIL = N - PCH * CHUNK


@functools.partial(
    pl.kernel,
    mesh=_mesh,
    out_type=jax.ShapeDtypeStruct((NC, G, H), jnp.float32),
    scratch_types=[
        pltpu.VMEM((CHUNK,), jnp.int32),
        pltpu.VMEM((CHUNK, H), jnp.float32),
        pltpu.VMEM((PTAIL,), jnp.int32),
        pltpu.VMEM((PTAIL, H), jnp.float32),
        pltpu.VMEM_SHARED((G, H), jnp.float32),
        pltpu.SemaphoreType.DMA,
    ],
)
def _pool(h_hbm, batch_hbm, out_hbm, bidx_v, rows_v, bidx_t, rows_t, acc, sem):
    cid = lax.axis_index("c")
    sid = lax.axis_index("s")
    wid = sid * NC + cid

    z16 = jnp.zeros((16,), jnp.float32)
    for r in range(CHUNK):
        for c in range(H // 16):
            rows_v[r, pl.ds(c * 16, 16)] = z16

    @pl.when(sid == 0)
    def _():
        pltpu.sync_copy(rows_v, acc)
    plsc.subcore_barrier()

    for rep in range(3):
        ch = wid + NW * rep

        @pl.when(ch < PCH)
        def _():
            off = pl.multiple_of(ch * CHUNK, 8)
            pltpu.sync_copy(batch_hbm.at[pl.ds(off, CHUNK)], bidx_v)
            pltpu.sync_copy(h_hbm.at[pl.ds(off, CHUNK)], rows_v)
            pltpu.sync_copy(rows_v, acc.at[bidx_v], add=True)

    @pl.when(wid == NW - 1)
    def _():
        pltpu.sync_copy(batch_hbm.at[pl.ds(PCH * CHUNK, PTAIL)], bidx_t)
        pltpu.sync_copy(h_hbm.at[pl.ds(PCH * CHUNK, PTAIL)], rows_t)
        pltpu.sync_copy(rows_t, acc.at[bidx_t], add=True)
    plsc.subcore_barrier()

    @pl.when(sid == 0)
    def _():
        pltpu.sync_copy(acc, rows_v)
        pltpu.sync_copy(rows_v, out_hbm.at[cid])


def _readout_body(p_ref, wo_ref, bo_ref, o_ref):
    pooled = p_ref[0] + p_ref[1]
    o_ref[...] = jnp.dot(pooled, wo_ref[...],
                         preferred_element_type=jnp.float32) + bo_ref[...]


_readout = pl.pallas_call(
    _readout_body,
    grid=(1,),
    in_specs=[
        pl.BlockSpec((NC, G, H), lambda i: (0, 0, 0)),
        pl.BlockSpec((H, 1), lambda i: (0, 0)),
        pl.BlockSpec((1, 1), lambda i: (0, 0)),
    ],
    out_specs=pl.BlockSpec((G, 1), lambda i: (0, 0)),
    out_shape=jax.ShapeDtypeStruct((G, 1), jnp.float32),
)


def kernel(x, edge_index, batch, W_enc, b_enc, W_out, b_out,
           W_l0, b_l0, W_r0, W_l1, b_l1, W_r1, W_l2, b_l2, W_r2):
    ei = edge_index.reshape(2 * E)
    deg = _degrees(edge_index[1].reshape(E, 1))
    deg = deg.reshape(DROWS * 128)[:N].reshape(N, 1)
    h = _enc(x, W_enc, b_enc.reshape(1, H))
    for (W_l, b_l, W_r) in ((W_l0, b_l0, W_r0), (W_l1, b_l1, W_r1)):
        aggp = _msgpass(h, ei)
        h = _combine(aggp, deg, h, W_l, b_l.reshape(1, H), W_r)
    aggp = _msgpass(h, ei)
    h3 = _combine(aggp, deg, h, W_l2, b_l2.reshape(1, H), W_r2)
    pooled = _pool(h3, batch)
    return _readout(pooled, W_out, b_out.reshape(1, 1))
